# Initial kernel scaffold; baseline (speedup 1.0000x reference)
import jax
import jax.numpy as jnp
from jax.experimental import pallas as pl


def kernel(A_s, A_t, gamma):
    raise NotImplementedError("write your pallas kernel here")


# speedup vs baseline: 1.6426x; 1.6426x over previous
import jax
import jax.numpy as jnp
from jax.experimental import pallas as pl
from jax.experimental.pallas import tpu as pltpu


def _fuse_norm_kernel(gamma_ref, s_ref, t_ref, o_ref):
    alpha = jax.nn.sigmoid(gamma_ref[0])
    af = alpha * s_ref[...] + (1.0 - alpha) * t_ref[...]
    row_sum = jnp.sum(af, axis=1, keepdims=True)
    row_sum = jnp.where(row_sum == 0.0, 1.0, row_sum)
    o_ref[...] = af / row_sum


def kernel(A_s, A_t, gamma):
    n, m = A_s.shape
    block_rows = 256
    grid = (n // block_rows,)
    gamma_arr = jnp.reshape(gamma, (1,)).astype(jnp.float32)
    return pl.pallas_call(
        _fuse_norm_kernel,
        grid=grid,
        in_specs=[
            pl.BlockSpec(memory_space=pltpu.SMEM),
            pl.BlockSpec((block_rows, m), lambda i: (i, 0)),
            pl.BlockSpec((block_rows, m), lambda i: (i, 0)),
        ],
        out_specs=pl.BlockSpec((block_rows, m), lambda i: (i, 0)),
        out_shape=jax.ShapeDtypeStruct((n, m), jnp.float32),
    )(gamma_arr, A_s, A_t)


# speedup vs baseline: 1.6491x; 1.0040x over previous
import jax
import jax.numpy as jnp
from jax.experimental import pallas as pl
from jax.experimental.pallas import tpu as pltpu


def _fuse_norm_kernel(gamma_ref, s_ref, t_ref, o_ref):
    alpha = jax.nn.sigmoid(gamma_ref[0])
    af = alpha * s_ref[...] + (1.0 - alpha) * t_ref[...]
    row_sum = jnp.sum(af, axis=1, keepdims=True)
    row_sum = jnp.where(row_sum == 0.0, 1.0, row_sum)
    o_ref[...] = af * (1.0 / row_sum)


def kernel(A_s, A_t, gamma):
    n, m = A_s.shape
    block_rows = 256
    grid = (n // block_rows,)
    gamma_arr = jnp.reshape(gamma, (1,)).astype(jnp.float32)
    return pl.pallas_call(
        _fuse_norm_kernel,
        grid=grid,
        in_specs=[
            pl.BlockSpec(memory_space=pltpu.SMEM),
            pl.BlockSpec((block_rows, m), lambda i: (i, 0)),
            pl.BlockSpec((block_rows, m), lambda i: (i, 0)),
        ],
        out_specs=pl.BlockSpec((block_rows, m), lambda i: (i, 0)),
        out_shape=jax.ShapeDtypeStruct((n, m), jnp.float32),
    )(gamma_arr, A_s, A_t)
